# Initial kernel scaffold; baseline (speedup 1.0000x reference)
import jax
import jax.numpy as jnp
from jax.experimental import pallas as pl


def kernel(x, edge_index, W1, b1, W2, b2, W3, b3):
    raise NotImplementedError("write your pallas kernel here")


# speedup vs baseline: 10.8805x; 10.8805x over previous
import functools

import jax
import jax.numpy as jnp
from jax import lax
from jax.experimental import pallas as pl
from jax.experimental.pallas import tpu as pltpu
from jax.experimental.pallas import tpu_sc as plsc

N = 10000
E = 320000
D = 128
NC = 2
NS = 16
NW = NC * NS
EPT = E // NW
K = 80
NCHUNK = EPT // K
RPT = N // NS
ZR = 125
DEGW = 16
L = 16

_mesh = plsc.VectorSubcoreMesh(core_axis_name="c", subcore_axis_name="s")


def _fill(ref, rows, width, value):
    def body(i, _):
        r = i // (width // L)
        j = i % (width // L)
        ref[r, pl.ds(j * L, L)] = jnp.full((L,), value, jnp.float32)
        return 0

    lax.fori_loop(0, rows * (width // L), body, 0)


def _deg_body(dst_hbm, out_hbm, dst_v, ones_v, zbuf, deg_sh):
    c = lax.axis_index("c")
    s = lax.axis_index("s")
    wid = c * NS + s
    _fill(ones_v, K, DEGW, 1.0)
    _fill(zbuf, ZR, DEGW, 0.0)
    row0 = s * RPT
    for t in range(RPT // ZR):
        pltpu.sync_copy(zbuf, deg_sh.at[pl.ds(row0 + t * ZR, ZR)])
    plsc.subcore_barrier()
    ebase = wid * EPT

    def chunk(i, _):
        pltpu.sync_copy(dst_hbm.at[pl.ds(ebase + i * K, K)], dst_v)
        pltpu.sync_copy(ones_v, deg_sh.at[dst_v], add=True)
        return 0

    lax.fori_loop(0, NCHUNK, chunk, 0)
    plsc.subcore_barrier()
    pltpu.sync_copy(deg_sh.at[pl.ds(row0, RPT)], out_hbm.at[c, s])


_deg_call = functools.partial(
    pl.kernel,
    out_type=jax.ShapeDtypeStruct((NC, NS, RPT, DEGW), jnp.float32),
    mesh=_mesh,
    scratch_types=[
        pltpu.VMEM((K,), jnp.int32),
        pltpu.VMEM((K, DEGW), jnp.float32),
        pltpu.VMEM((ZR, DEGW), jnp.float32),
        pltpu.VMEM_SHARED((N, DEGW), jnp.float32),
    ],
)(_deg_body)


def _agg_body(p_hbm, src_hbm, dst_hbm, out_hbm, src_v, dst_v, rows_v, zbuf,
              acc_sh, sem):
    c = lax.axis_index("c")
    s = lax.axis_index("s")
    wid = c * NS + s
    _fill(zbuf, ZR, D, 0.0)
    row0 = s * RPT
    for t in range(RPT // ZR):
        pltpu.sync_copy(zbuf, acc_sh.at[pl.ds(row0 + t * ZR, ZR)])
    plsc.subcore_barrier()
    ebase = wid * EPT

    def chunk(i, _):
        off = ebase + i * K
        pltpu.sync_copy(src_hbm.at[pl.ds(off, K)], src_v)
        pltpu.sync_copy(dst_hbm.at[pl.ds(off, K)], dst_v)
        pltpu.async_copy(p_hbm.at[src_v], rows_v, sem).wait()
        pltpu.sync_copy(rows_v, acc_sh.at[dst_v], add=True)
        return 0

    lax.fori_loop(0, NCHUNK, chunk, 0)
    plsc.subcore_barrier()
    pltpu.sync_copy(acc_sh.at[pl.ds(row0, RPT)], out_hbm.at[c, s])


_agg_call = functools.partial(
    pl.kernel,
    out_type=jax.ShapeDtypeStruct((NC, NS, RPT, D), jnp.float32),
    mesh=_mesh,
    scratch_types=[
        pltpu.VMEM((K,), jnp.int32),
        pltpu.VMEM((K,), jnp.int32),
        pltpu.VMEM((K, D), jnp.float32),
        pltpu.VMEM((ZR, D), jnp.float32),
        pltpu.VMEM_SHARED((N, D), jnp.float32),
        pltpu.SemaphoreType.DMA,
    ],
)(_agg_body)


BM = 1000


def _tc1_body(x_ref, w_ref, degp_ref, p_ref, dinv_ref):
    deg = degp_ref[0] + degp_ref[1] + 1.0
    dinv = lax.rsqrt(deg)
    dinv_ref[...] = dinv
    dcol = dinv[:, 0:1]
    p_ref[...] = jnp.dot(x_ref[...], w_ref[...],
                         preferred_element_type=jnp.float32) * dcol


_tc1_call = pl.pallas_call(
    _tc1_body,
    grid=(N // BM,),
    in_specs=[
        pl.BlockSpec((BM, D), lambda i: (i, 0)),
        pl.BlockSpec((D, D), lambda i: (0, 0)),
        pl.BlockSpec((NC, BM, DEGW), lambda i: (0, i, 0)),
    ],
    out_specs=[
        pl.BlockSpec((BM, D), lambda i: (i, 0)),
        pl.BlockSpec((BM, DEGW), lambda i: (i, 0)),
    ],
    out_shape=[
        jax.ShapeDtypeStruct((N, D), jnp.float32),
        jax.ShapeDtypeStruct((N, DEGW), jnp.float32),
    ],
)


def _tcmid_body(sp_ref, p_ref, dinv_ref, b_ref, w_ref, out_ref):
    dcol = dinv_ref[:, 0:1]
    t = (sp_ref[0] + sp_ref[1] + p_ref[...]) * dcol + b_ref[...]
    t = jnp.maximum(t, 0.0)
    out_ref[...] = jnp.dot(t, w_ref[...],
                           preferred_element_type=jnp.float32) * dcol


_tcmid_call = pl.pallas_call(
    _tcmid_body,
    grid=(N // BM,),
    in_specs=[
        pl.BlockSpec((NC, BM, D), lambda i: (0, i, 0)),
        pl.BlockSpec((BM, D), lambda i: (i, 0)),
        pl.BlockSpec((BM, DEGW), lambda i: (i, 0)),
        pl.BlockSpec((1, D), lambda i: (0, 0)),
        pl.BlockSpec((D, D), lambda i: (0, 0)),
    ],
    out_specs=pl.BlockSpec((BM, D), lambda i: (i, 0)),
    out_shape=jax.ShapeDtypeStruct((N, D), jnp.float32),
)


def _tcfin_body(sp_ref, p_ref, dinv_ref, b_ref, out_ref):
    dcol = dinv_ref[:, 0:1]
    out_ref[...] = (sp_ref[0] + sp_ref[1] + p_ref[...]) * dcol + b_ref[...]


_tcfin_call = pl.pallas_call(
    _tcfin_body,
    grid=(N // BM,),
    in_specs=[
        pl.BlockSpec((NC, BM, D), lambda i: (0, i, 0)),
        pl.BlockSpec((BM, D), lambda i: (i, 0)),
        pl.BlockSpec((BM, DEGW), lambda i: (i, 0)),
        pl.BlockSpec((1, D), lambda i: (0, 0)),
    ],
    out_specs=pl.BlockSpec((BM, D), lambda i: (i, 0)),
    out_shape=jax.ShapeDtypeStruct((N, D), jnp.float32),
)


def kernel(x, edge_index, W1, b1, W2, b2, W3, b3):
    src = edge_index[0]
    dst = edge_index[1]
    degp = _deg_call(dst).reshape(NC, N, DEGW)
    p1, dinv16 = _tc1_call(x, W1, degp)
    sp1 = _agg_call(p1, src, dst).reshape(NC, N, D)
    p2 = _tcmid_call(sp1, p1, dinv16, b1.reshape(1, D), W2)
    sp2 = _agg_call(p2, src, dst).reshape(NC, N, D)
    p3 = _tcmid_call(sp2, p2, dinv16, b2.reshape(1, D), W3)
    sp3 = _agg_call(p3, src, dst).reshape(NC, N, D)
    return _tcfin_call(sp3, p3, dinv16, b3.reshape(1, D))


# speedup vs baseline: 22.2595x; 2.0458x over previous
import functools

import jax
import jax.numpy as jnp
from jax import lax
from jax.experimental import pallas as pl
from jax.experimental.pallas import tpu as pltpu
from jax.experimental.pallas import tpu_sc as plsc

N = 10000
E = 320000
D = 128
NC = 2
NS = 16
NW = NC * NS
EPT = E // NW
K = 80
NCHUNK = EPT // K
RPT = N // NS
ZR = 125
DEGW = 16
L = 16

_mesh = plsc.VectorSubcoreMesh(core_axis_name="c", subcore_axis_name="s")


def _fill(ref, rows, width, value):
    def body(i, _):
        r = i // (width // L)
        j = i % (width // L)
        ref[r, pl.ds(j * L, L)] = jnp.full((L,), value, jnp.float32)
        return 0

    lax.fori_loop(0, rows * (width // L), body, 0)


def _deg_body(dst_hbm, out_hbm, dst_v, ones_v, zbuf, deg_sh):
    c = lax.axis_index("c")
    s = lax.axis_index("s")
    wid = c * NS + s
    _fill(ones_v, K, DEGW, 1.0)
    _fill(zbuf, ZR, DEGW, 0.0)
    row0 = s * RPT
    for t in range(RPT // ZR):
        pltpu.sync_copy(zbuf, deg_sh.at[pl.ds(row0 + t * ZR, ZR)])
    plsc.subcore_barrier()
    ebase = wid * EPT

    def chunk(i, _):
        pltpu.sync_copy(dst_hbm.at[pl.ds(ebase + i * K, K)], dst_v)
        pltpu.sync_copy(ones_v, deg_sh.at[dst_v], add=True)
        return 0

    lax.fori_loop(0, NCHUNK, chunk, 0)
    plsc.subcore_barrier()
    pltpu.sync_copy(deg_sh.at[pl.ds(row0, RPT)], out_hbm.at[c, s])


_deg_call = functools.partial(
    pl.kernel,
    out_type=jax.ShapeDtypeStruct((NC, NS, RPT, DEGW), jnp.float32),
    mesh=_mesh,
    scratch_types=[
        pltpu.VMEM((K,), jnp.int32),
        pltpu.VMEM((K, DEGW), jnp.float32),
        pltpu.VMEM((ZR, DEGW), jnp.float32),
        pltpu.VMEM_SHARED((N, DEGW), jnp.float32),
    ],
)(_deg_body)


def _agg_body(p_hbm, pk_hbm, out_hbm, pk, sidx2, didx2, buf0, buf1,
              acc_sh, gs0, gs1):
    c = lax.axis_index("c")
    s = lax.axis_index("s")
    wid = c * NS + s
    pltpu.sync_copy(pk_hbm.at[pl.ds(wid * EPT, EPT)], pk)

    def unpack(i, slot):
        for j in range(K // L):
            v = pk[pl.ds(i * K + j * L, L)]
            sidx2[slot, pl.ds(j * L, L)] = lax.shift_right_logical(v, 14)
            didx2[slot, pl.ds(j * L, L)] = lax.bitwise_and(v, 16383)

    _fill(buf0, K, D, 0.0)
    row0 = s * RPT
    for t in range(RPT // K):
        pltpu.sync_copy(buf0, acc_sh.at[pl.ds(row0 + t * K, K)])
    rem = RPT % K
    if rem:
        pltpu.sync_copy(buf0.at[pl.ds(0, rem)],
                        acc_sh.at[pl.ds(row0 + (RPT // K) * K, rem)])
    plsc.subcore_barrier()

    unpack(0, 0)
    pltpu.async_copy(p_hbm.at[sidx2.at[0]], buf0, gs0)

    def pair(i2, _):
        a = 2 * i2
        unpack(a + 1, 1)
        pltpu.async_copy(p_hbm.at[sidx2.at[1]], buf1, gs1)
        pltpu.make_async_copy(p_hbm.at[sidx2.at[0]], buf0, gs0).wait()
        pltpu.sync_copy(buf0, acc_sh.at[didx2.at[0]], add=True)
        unpack(a + 2, 0)
        pltpu.async_copy(p_hbm.at[sidx2.at[0]], buf0, gs0)
        pltpu.make_async_copy(p_hbm.at[sidx2.at[1]], buf1, gs1).wait()
        pltpu.sync_copy(buf1, acc_sh.at[didx2.at[1]], add=True)
        return 0

    lax.fori_loop(0, (NCHUNK - 1) // 2, pair, 0)
    pltpu.make_async_copy(p_hbm.at[sidx2.at[0]], buf0, gs0).wait()
    pltpu.sync_copy(buf0, acc_sh.at[didx2.at[0]], add=True)
    plsc.subcore_barrier()
    pltpu.sync_copy(acc_sh.at[pl.ds(row0, RPT)], out_hbm.at[c, s])


_agg_call = functools.partial(
    pl.kernel,
    out_type=jax.ShapeDtypeStruct((NC, NS, RPT, D), jnp.float32),
    mesh=_mesh,
    scratch_types=[
        pltpu.VMEM((EPT,), jnp.int32),
        pltpu.VMEM((2, K), jnp.int32),
        pltpu.VMEM((2, K), jnp.int32),
        pltpu.VMEM((K, D), jnp.float32),
        pltpu.VMEM((K, D), jnp.float32),
        pltpu.VMEM_SHARED((N, D), jnp.float32),
        pltpu.SemaphoreType.DMA,
        pltpu.SemaphoreType.DMA,
    ],
)(_agg_body)


BM = 1000


def _tc1_body(x_ref, w_ref, degp_ref, p_ref, dinv_ref):
    deg = degp_ref[0] + degp_ref[1] + 1.0
    dinv = lax.rsqrt(deg)
    dinv_ref[...] = dinv
    dcol = dinv[:, 0:1]
    p_ref[...] = jnp.dot(x_ref[...], w_ref[...],
                         preferred_element_type=jnp.float32) * dcol


_tc1_call = pl.pallas_call(
    _tc1_body,
    grid=(N // BM,),
    in_specs=[
        pl.BlockSpec((BM, D), lambda i: (i, 0)),
        pl.BlockSpec((D, D), lambda i: (0, 0)),
        pl.BlockSpec((NC, BM, DEGW), lambda i: (0, i, 0)),
    ],
    out_specs=[
        pl.BlockSpec((BM, D), lambda i: (i, 0)),
        pl.BlockSpec((BM, DEGW), lambda i: (i, 0)),
    ],
    out_shape=[
        jax.ShapeDtypeStruct((N, D), jnp.float32),
        jax.ShapeDtypeStruct((N, DEGW), jnp.float32),
    ],
)


def _tcmid_body(sp_ref, p_ref, dinv_ref, b_ref, w_ref, out_ref):
    dcol = dinv_ref[:, 0:1]
    t = (sp_ref[0] + sp_ref[1] + p_ref[...]) * dcol + b_ref[...]
    t = jnp.maximum(t, 0.0)
    out_ref[...] = jnp.dot(t, w_ref[...],
                           preferred_element_type=jnp.float32) * dcol


_tcmid_call = pl.pallas_call(
    _tcmid_body,
    grid=(N // BM,),
    in_specs=[
        pl.BlockSpec((NC, BM, D), lambda i: (0, i, 0)),
        pl.BlockSpec((BM, D), lambda i: (i, 0)),
        pl.BlockSpec((BM, DEGW), lambda i: (i, 0)),
        pl.BlockSpec((1, D), lambda i: (0, 0)),
        pl.BlockSpec((D, D), lambda i: (0, 0)),
    ],
    out_specs=pl.BlockSpec((BM, D), lambda i: (i, 0)),
    out_shape=jax.ShapeDtypeStruct((N, D), jnp.float32),
)


def _tcfin_body(sp_ref, p_ref, dinv_ref, b_ref, out_ref):
    dcol = dinv_ref[:, 0:1]
    out_ref[...] = (sp_ref[0] + sp_ref[1] + p_ref[...]) * dcol + b_ref[...]


_tcfin_call = pl.pallas_call(
    _tcfin_body,
    grid=(N // BM,),
    in_specs=[
        pl.BlockSpec((NC, BM, D), lambda i: (0, i, 0)),
        pl.BlockSpec((BM, D), lambda i: (i, 0)),
        pl.BlockSpec((BM, DEGW), lambda i: (i, 0)),
        pl.BlockSpec((1, D), lambda i: (0, 0)),
    ],
    out_specs=pl.BlockSpec((BM, D), lambda i: (i, 0)),
    out_shape=jax.ShapeDtypeStruct((N, D), jnp.float32),
)


def kernel(x, edge_index, W1, b1, W2, b2, W3, b3):
    dst = edge_index[1]
    pk1 = (edge_index[0] << 14) | edge_index[1]
    degp = _deg_call(dst).reshape(NC, N, DEGW)
    p1, dinv16 = _tc1_call(x, W1, degp)
    sp1 = _agg_call(p1, pk1).reshape(NC, N, D)
    p2 = _tcmid_call(sp1, p1, dinv16, b1.reshape(1, D), W2)
    sp2 = _agg_call(p2, pk1).reshape(NC, N, D)
    p3 = _tcmid_call(sp2, p2, dinv16, b2.reshape(1, D), W3)
    sp3 = _agg_call(p3, pk1).reshape(NC, N, D)
    return _tcfin_call(sp3, p3, dinv16, b3.reshape(1, D))


# speedup vs baseline: 25.7106x; 1.1550x over previous
import functools

import jax
import jax.numpy as jnp
from jax import lax
from jax.experimental import pallas as pl
from jax.experimental.pallas import tpu as pltpu
from jax.experimental.pallas import tpu_sc as plsc

N = 10000
E = 320000
D = 128
NC = 2
NS = 16
NW = NC * NS
EPT = E // NW
K = 80
NCHUNK = EPT // K
RPT = N // NS
ZR = 125
DEGW = 16
L = 16

_mesh = plsc.VectorSubcoreMesh(core_axis_name="c", subcore_axis_name="s")


def _fill(ref, rows, width, value):
    def body(i, _):
        r = i // (width // L)
        j = i % (width // L)
        ref[r, pl.ds(j * L, L)] = jnp.full((L,), value, jnp.float32)
        return 0

    lax.fori_loop(0, rows * (width // L), body, 0)


PROW = N // L


def _deg_body(dst_hbm, out_hbm, dv, pdeg):
    c = lax.axis_index("c")
    s = lax.axis_index("s")
    wid = c * NS + s
    pltpu.sync_copy(dst_hbm.at[pl.ds(wid * EPT, EPT)], dv)

    def z(i, _):
        pdeg[pl.ds(i * L, L)] = jnp.zeros((L,), jnp.float32)
        return 0

    lax.fori_loop(0, N // L, z, 0)
    ones = jnp.ones((L,), jnp.float32)

    def acc(i, _):
        idx = dv[pl.ds(i * L, L)]
        plsc.addupdate_scatter(pdeg, [idx], ones)
        return 0

    lax.fori_loop(0, EPT // L, acc, 0)
    pltpu.sync_copy(pdeg, out_hbm.at[c, s])


_deg_call = functools.partial(
    pl.kernel,
    out_type=jax.ShapeDtypeStruct((NC, NS, N), jnp.float32),
    mesh=_mesh,
    compiler_params=pltpu.CompilerParams(needs_layout_passes=False),
    scratch_types=[
        pltpu.VMEM((EPT,), jnp.int32),
        pltpu.VMEM((N,), jnp.float32),
    ],
)(_deg_body)


def _agg_body(p_hbm, pk_hbm, out_hbm, pk, sidx2, didx2, buf0, buf1,
              acc_sh, gs0, gs1):
    c = lax.axis_index("c")
    s = lax.axis_index("s")
    wid = c * NS + s
    pltpu.sync_copy(pk_hbm.at[pl.ds(wid * EPT, EPT)], pk)

    def unpack(i, slot):
        for j in range(K // L):
            v = pk[pl.ds(i * K + j * L, L)]
            sidx2[slot, pl.ds(j * L, L)] = lax.shift_right_logical(v, 14)
            didx2[slot, pl.ds(j * L, L)] = lax.bitwise_and(v, 16383)

    _fill(buf0, K, D, 0.0)
    row0 = s * RPT
    for t in range(RPT // K):
        pltpu.sync_copy(buf0, acc_sh.at[pl.ds(row0 + t * K, K)])
    rem = RPT % K
    if rem:
        pltpu.sync_copy(buf0.at[pl.ds(0, rem)],
                        acc_sh.at[pl.ds(row0 + (RPT // K) * K, rem)])
    plsc.subcore_barrier()

    unpack(0, 0)
    pltpu.async_copy(p_hbm.at[sidx2.at[0]], buf0, gs0)

    def pair(i2, _):
        a = 2 * i2
        unpack(a + 1, 1)
        pltpu.async_copy(p_hbm.at[sidx2.at[1]], buf1, gs1)
        pltpu.make_async_copy(p_hbm.at[sidx2.at[0]], buf0, gs0).wait()
        pltpu.sync_copy(buf0, acc_sh.at[didx2.at[0]], add=True)
        unpack(a + 2, 0)
        pltpu.async_copy(p_hbm.at[sidx2.at[0]], buf0, gs0)
        pltpu.make_async_copy(p_hbm.at[sidx2.at[1]], buf1, gs1).wait()
        pltpu.sync_copy(buf1, acc_sh.at[didx2.at[1]], add=True)
        return 0

    lax.fori_loop(0, (NCHUNK - 1) // 2, pair, 0)
    pltpu.make_async_copy(p_hbm.at[sidx2.at[0]], buf0, gs0).wait()
    pltpu.sync_copy(buf0, acc_sh.at[didx2.at[0]], add=True)
    plsc.subcore_barrier()
    pltpu.sync_copy(acc_sh.at[pl.ds(row0, RPT)], out_hbm.at[c, s])


_agg_call = functools.partial(
    pl.kernel,
    out_type=jax.ShapeDtypeStruct((NC, NS, RPT, D), jnp.float32),
    mesh=_mesh,
    scratch_types=[
        pltpu.VMEM((EPT,), jnp.int32),
        pltpu.VMEM((2, K), jnp.int32),
        pltpu.VMEM((2, K), jnp.int32),
        pltpu.VMEM((K, D), jnp.float32),
        pltpu.VMEM((K, D), jnp.float32),
        pltpu.VMEM_SHARED((N, D), jnp.float32),
        pltpu.SemaphoreType.DMA,
        pltpu.SemaphoreType.DMA,
    ],
)(_agg_body)


BM = 1000


def _tc1_body(x_ref, w_ref, degp_ref, p_ref, dinv_ref):
    deg = jnp.sum(degp_ref[...], axis=1, keepdims=True) + 1.0
    dcol = lax.rsqrt(deg)
    dinv_ref[...] = jnp.broadcast_to(dcol, (BM, DEGW))
    p_ref[...] = jnp.dot(x_ref[...], w_ref[...],
                         preferred_element_type=jnp.float32) * dcol


_tc1_call = pl.pallas_call(
    _tc1_body,
    grid=(N // BM,),
    in_specs=[
        pl.BlockSpec((BM, D), lambda i: (i, 0)),
        pl.BlockSpec((D, D), lambda i: (0, 0)),
        pl.BlockSpec((BM, NW), lambda i: (i, 0)),
    ],
    out_specs=[
        pl.BlockSpec((BM, D), lambda i: (i, 0)),
        pl.BlockSpec((BM, DEGW), lambda i: (i, 0)),
    ],
    out_shape=[
        jax.ShapeDtypeStruct((N, D), jnp.float32),
        jax.ShapeDtypeStruct((N, DEGW), jnp.float32),
    ],
)


def _tcmid_body(sp_ref, p_ref, dinv_ref, b_ref, w_ref, out_ref):
    dcol = dinv_ref[:, 0:1]
    t = (sp_ref[0] + sp_ref[1] + p_ref[...]) * dcol + b_ref[...]
    t = jnp.maximum(t, 0.0)
    out_ref[...] = jnp.dot(t, w_ref[...],
                           preferred_element_type=jnp.float32) * dcol


_tcmid_call = pl.pallas_call(
    _tcmid_body,
    grid=(N // BM,),
    in_specs=[
        pl.BlockSpec((NC, BM, D), lambda i: (0, i, 0)),
        pl.BlockSpec((BM, D), lambda i: (i, 0)),
        pl.BlockSpec((BM, DEGW), lambda i: (i, 0)),
        pl.BlockSpec((1, D), lambda i: (0, 0)),
        pl.BlockSpec((D, D), lambda i: (0, 0)),
    ],
    out_specs=pl.BlockSpec((BM, D), lambda i: (i, 0)),
    out_shape=jax.ShapeDtypeStruct((N, D), jnp.float32),
)


def _tcfin_body(sp_ref, p_ref, dinv_ref, b_ref, out_ref):
    dcol = dinv_ref[:, 0:1]
    out_ref[...] = (sp_ref[0] + sp_ref[1] + p_ref[...]) * dcol + b_ref[...]


_tcfin_call = pl.pallas_call(
    _tcfin_body,
    grid=(N // BM,),
    in_specs=[
        pl.BlockSpec((NC, BM, D), lambda i: (0, i, 0)),
        pl.BlockSpec((BM, D), lambda i: (i, 0)),
        pl.BlockSpec((BM, DEGW), lambda i: (i, 0)),
        pl.BlockSpec((1, D), lambda i: (0, 0)),
    ],
    out_specs=pl.BlockSpec((BM, D), lambda i: (i, 0)),
    out_shape=jax.ShapeDtypeStruct((N, D), jnp.float32),
)


def kernel(x, edge_index, W1, b1, W2, b2, W3, b3):
    dst = edge_index[1]
    pk1 = (edge_index[0] << 14) | edge_index[1]
    degp = _deg_call(dst).reshape(NW, N).T
    p1, dinv16 = _tc1_call(x, W1, degp)
    sp1 = _agg_call(p1, pk1).reshape(NC, N, D)
    p2 = _tcmid_call(sp1, p1, dinv16, b1.reshape(1, D), W2)
    sp2 = _agg_call(p2, pk1).reshape(NC, N, D)
    p3 = _tcmid_call(sp2, p2, dinv16, b2.reshape(1, D), W3)
    sp3 = _agg_call(p3, pk1).reshape(NC, N, D)
    return _tcfin_call(sp3, p3, dinv16, b3.reshape(1, D))
